# Initial kernel scaffold; baseline (speedup 1.0000x reference)
import jax
import jax.numpy as jnp
from jax.experimental import pallas as pl


def kernel(x, query_w, query_b, gate_w, gate_b, mix_logit, conf_w1, conf_b1, conf_w2, conf_b2, fast_keys, fast_vals, deep_keys, deep_vals):
    raise NotImplementedError("write your pallas kernel here")


# speedup vs baseline: 1.2849x; 1.2849x over previous
import jax
import jax.numpy as jnp
from jax.experimental import pallas as pl
from jax.experimental.pallas import tpu as pltpu

_BLOCK_B = 512


def _fused_body(x_ref, qw_ref, qb_ref, gw_ref, gbm_ref, cw1_ref, cb1_ref,
                cw2_ref, cb2_ref, fk_ref, fv_ref, dk_ref, dv_ref, out_ref):
    xb = x_ref[:]

    q = jnp.dot(xb, qw_ref[:], preferred_element_type=jnp.float32) + qb_ref[:]
    qnorm = jnp.sqrt(jnp.sum(q * q, axis=1, keepdims=True))
    qn = (q * (1.0 / jnp.maximum(qnorm, 1e-12))).astype(jnp.bfloat16)

    def tier(k_ref, v_ref):
        k = k_ref[:]
        kf = k.astype(jnp.float32)
        knorm = jnp.sqrt(jnp.sum(kf * kf, axis=1))
        kinv = 1.0 / jnp.maximum(knorm, 1e-12)
        s = jax.lax.dot_general(qn, k, (((1,), (1,)), ((), ())),
                                preferred_element_type=jnp.float32)
        s = s * kinv[None, :]
        m = jnp.max(s, axis=1, keepdims=True)
        e = jnp.exp(s - m)
        w = (e * (1.0 / jnp.sum(e, axis=1, keepdims=True))).astype(jnp.bfloat16)
        return jnp.dot(w, v_ref[:], preferred_element_type=jnp.float32)

    fast_out = tier(fk_ref, fv_ref)
    deep_out = tier(dk_ref, dv_ref)

    xf = xb.astype(jnp.float32)
    g = jnp.sum(xf * gw_ref[:], axis=1, keepdims=True) + gbm_ref[:]
    gate = jax.nn.sigmoid(g)

    h = jax.nn.relu(jnp.dot(xb, cw1_ref[:], preferred_element_type=jnp.float32)
                    + cb1_ref[:])
    c = jnp.sum(h * cw2_ref[:], axis=1, keepdims=True) + cb2_ref[:]
    conf = jax.nn.sigmoid(c)

    out_ref[:] = (gate * fast_out + (1.0 - gate) * deep_out) * conf


def kernel(x, query_w, query_b, gate_w, gate_b, mix_logit, conf_w1, conf_b1,
           conf_w2, conf_b2, fast_keys, fast_vals, deep_keys, deep_vals):
    B, D = x.shape
    M = fast_keys.shape[0]
    H = conf_w1.shape[1]
    bB = _BLOCK_B

    bf16 = jnp.bfloat16
    x16 = x.astype(bf16)
    qw16 = query_w.astype(bf16)
    cw1_16 = conf_w1.astype(bf16)
    fk16 = fast_keys.astype(bf16)
    fv16 = fast_vals.astype(bf16)
    dk16 = deep_keys.astype(bf16)
    dv16 = deep_vals.astype(bf16)

    qb2 = query_b.reshape(1, D)
    gw2 = gate_w.reshape(1, D)
    gbm = (gate_b + mix_logit).reshape(1, 1)
    cb1_2 = conf_b1.reshape(1, H)
    cw2_2 = conf_w2.reshape(1, H)
    cb2_2 = conf_b2.reshape(1, 1)

    full = lambda i: (0, 0)
    row = lambda i: (i, 0)

    out = pl.pallas_call(
        _fused_body,
        grid=(B // bB,),
        in_specs=[
            pl.BlockSpec((bB, D), row),
            pl.BlockSpec((D, D), full),
            pl.BlockSpec((1, D), full),
            pl.BlockSpec((1, D), full),
            pl.BlockSpec((1, 1), full),
            pl.BlockSpec((D, H), full),
            pl.BlockSpec((1, H), full),
            pl.BlockSpec((1, H), full),
            pl.BlockSpec((1, 1), full),
            pl.BlockSpec((M, D), full),
            pl.BlockSpec((M, D), full),
            pl.BlockSpec((M, D), full),
            pl.BlockSpec((M, D), full),
        ],
        out_specs=pl.BlockSpec((bB, D), row),
        out_shape=jax.ShapeDtypeStruct((B, D), jnp.float32),
        compiler_params=pltpu.CompilerParams(
            dimension_semantics=("arbitrary",),
        ),
    )(x16, qw16, qb2, gw2, gbm, cw1_16, cb1_2, cw2_2, cb2_2,
      fk16, fv16, dk16, dv16)
    return out


# speedup vs baseline: 1.2859x; 1.0008x over previous
import jax
import jax.numpy as jnp
from jax.experimental import pallas as pl
from jax.experimental.pallas import tpu as pltpu

_BLOCK_B = 512


def _fused_body(x_ref, qw_ref, qb_ref, gw_ref, gbm_ref, cw1_ref, cb1_ref,
                cw2_ref, cb2_ref, fk_ref, fv_ref, dk_ref, dv_ref, out_ref):
    xb = x_ref[:]

    q = jnp.dot(xb, qw_ref[:], preferred_element_type=jnp.float32) + qb_ref[:]
    qnorm = jnp.sqrt(jnp.sum(q * q, axis=1, keepdims=True))
    qn = (q * (1.0 / jnp.maximum(qnorm, 1e-12))).astype(jnp.bfloat16)

    def tier(k_ref, v_ref):
        k = k_ref[:]
        kf = k.astype(jnp.float32)
        knorm = jnp.sqrt(jnp.sum(kf * kf, axis=1))
        kinv = 1.0 / jnp.maximum(knorm, 1e-12)
        s = jax.lax.dot_general(qn, k, (((1,), (1,)), ((), ())),
                                preferred_element_type=jnp.float32)
        s = s * kinv[None, :]
        m = jnp.max(s, axis=1, keepdims=True)
        e = jnp.exp(s - m)
        w = (e * (1.0 / jnp.sum(e, axis=1, keepdims=True))).astype(jnp.bfloat16)
        return jnp.dot(w, v_ref[:], preferred_element_type=jnp.float32)

    fast_out = tier(fk_ref, fv_ref)
    deep_out = tier(dk_ref, dv_ref)

    xf = xb.astype(jnp.float32)
    g = jnp.sum(xf * gw_ref[:], axis=1, keepdims=True) + gbm_ref[:]
    gate = jax.nn.sigmoid(g)

    h = jax.nn.relu(jnp.dot(xb, cw1_ref[:], preferred_element_type=jnp.float32)
                    + cb1_ref[:])
    c = jnp.sum(h * cw2_ref[:], axis=1, keepdims=True) + cb2_ref[:]
    conf = jax.nn.sigmoid(c)

    out_ref[:] = (gate * fast_out + (1.0 - gate) * deep_out) * conf


def kernel(x, query_w, query_b, gate_w, gate_b, mix_logit, conf_w1, conf_b1,
           conf_w2, conf_b2, fast_keys, fast_vals, deep_keys, deep_vals):
    B, D = x.shape
    M = fast_keys.shape[0]
    H = conf_w1.shape[1]
    bB = _BLOCK_B

    bf16 = jnp.bfloat16
    x16 = x.astype(bf16)
    qw16 = query_w.astype(bf16)
    cw1_16 = conf_w1.astype(bf16)
    fk16 = fast_keys.astype(bf16)
    fv16 = fast_vals.astype(bf16)
    dk16 = deep_keys.astype(bf16)
    dv16 = deep_vals.astype(bf16)

    qb2 = query_b.reshape(1, D)
    gw2 = gate_w.reshape(1, D)
    gbm = (gate_b + mix_logit).reshape(1, 1)
    cb1_2 = conf_b1.reshape(1, H)
    cw2_2 = conf_w2.reshape(1, H)
    cb2_2 = conf_b2.reshape(1, 1)

    full = lambda i: (0, 0)
    row = lambda i: (i, 0)

    out = pl.pallas_call(
        _fused_body,
        grid=(B // bB,),
        in_specs=[
            pl.BlockSpec((bB, D), row),
            pl.BlockSpec((D, D), full),
            pl.BlockSpec((1, D), full),
            pl.BlockSpec((1, D), full),
            pl.BlockSpec((1, 1), full),
            pl.BlockSpec((D, H), full),
            pl.BlockSpec((1, H), full),
            pl.BlockSpec((1, H), full),
            pl.BlockSpec((1, 1), full),
            pl.BlockSpec((M, D), full),
            pl.BlockSpec((M, D), full),
            pl.BlockSpec((M, D), full),
            pl.BlockSpec((M, D), full),
        ],
        out_specs=pl.BlockSpec((bB, D), row),
        out_shape=jax.ShapeDtypeStruct((B, D), jnp.float32),
        compiler_params=pltpu.CompilerParams(
            dimension_semantics=("parallel",),
        ),
    )(x16, qw16, qb2, gw2, gbm, cw1_16, cb1_2, cw2_2, cb2_2,
      fk16, fv16, dk16, dv16)
    return out


# speedup vs baseline: 1.6217x; 1.2611x over previous
import jax
import jax.numpy as jnp
from jax.experimental import pallas as pl
from jax.experimental.pallas import tpu as pltpu

_BLOCK_B = 512


def _inv_norm_rows(k):
    kf = k.astype(jnp.float32)
    n = jnp.sqrt(jnp.sum(kf * kf, axis=1, keepdims=True))
    return 1.0 / jnp.maximum(n, 1e-12)


def _fused_body(x_ref, qw_ref, qb_ref, gw_ref, gbm_ref, cw1_ref, cb1_ref,
                cw2_ref, cb2_ref, fk_ref, fv_ref, dk_ref, dv_ref, out_ref,
                qw16, cw116, fk16, fv16, dk16, dv16, fkinv, dkinv):
    @pl.when(pl.program_id(0) == 0)
    def _init():
        qw16[:] = qw_ref[:].astype(jnp.bfloat16)
        cw116[:] = cw1_ref[:].astype(jnp.bfloat16)
        fk16[:] = fk_ref[:].astype(jnp.bfloat16)
        fv16[:] = fv_ref[:].astype(jnp.bfloat16)
        dk16[:] = dk_ref[:].astype(jnp.bfloat16)
        dv16[:] = dv_ref[:].astype(jnp.bfloat16)
        fkinv[:] = _inv_norm_rows(fk_ref[:]).T
        dkinv[:] = _inv_norm_rows(dk_ref[:]).T

    xf = x_ref[:]
    xb = xf.astype(jnp.bfloat16)

    q = jnp.dot(xb, qw16[:], preferred_element_type=jnp.float32) + qb_ref[:]
    qnorm = jnp.sqrt(jnp.sum(q * q, axis=1, keepdims=True))
    qn = (q * (1.0 / jnp.maximum(qnorm, 1e-12))).astype(jnp.bfloat16)

    def tier(k16, v16, kinv):
        s = jax.lax.dot_general(qn, k16[:], (((1,), (1,)), ((), ())),
                                preferred_element_type=jnp.float32)
        s = s * kinv[:]
        m = jnp.max(s, axis=1, keepdims=True)
        e = jnp.exp(s - m)
        w = (e * (1.0 / jnp.sum(e, axis=1, keepdims=True))).astype(jnp.bfloat16)
        return jnp.dot(w, v16[:], preferred_element_type=jnp.float32)

    fast_out = tier(fk16, fv16, fkinv)
    deep_out = tier(dk16, dv16, dkinv)

    g = jnp.sum(xf * gw_ref[:], axis=1, keepdims=True) + gbm_ref[:]
    gate = jax.nn.sigmoid(g)

    h = jax.nn.relu(jnp.dot(xb, cw116[:], preferred_element_type=jnp.float32)
                    + cb1_ref[:])
    c = jnp.sum(h * cw2_ref[:], axis=1, keepdims=True) + cb2_ref[:]
    conf = jax.nn.sigmoid(c)

    out_ref[:] = (gate * fast_out + (1.0 - gate) * deep_out) * conf


def kernel(x, query_w, query_b, gate_w, gate_b, mix_logit, conf_w1, conf_b1,
           conf_w2, conf_b2, fast_keys, fast_vals, deep_keys, deep_vals):
    B, D = x.shape
    M = fast_keys.shape[0]
    H = conf_w1.shape[1]
    bB = _BLOCK_B

    qb2 = query_b.reshape(1, D)
    gw2 = gate_w.reshape(1, D)
    gbm = (gate_b + mix_logit).reshape(1, 1)
    cb1_2 = conf_b1.reshape(1, H)
    cw2_2 = conf_w2.reshape(1, H)
    cb2_2 = conf_b2.reshape(1, 1)

    full = lambda i: (0, 0)
    row = lambda i: (i, 0)

    out = pl.pallas_call(
        _fused_body,
        grid=(B // bB,),
        in_specs=[
            pl.BlockSpec((bB, D), row),
            pl.BlockSpec((D, D), full),
            pl.BlockSpec((1, D), full),
            pl.BlockSpec((1, D), full),
            pl.BlockSpec((1, 1), full),
            pl.BlockSpec((D, H), full),
            pl.BlockSpec((1, H), full),
            pl.BlockSpec((1, H), full),
            pl.BlockSpec((1, 1), full),
            pl.BlockSpec((M, D), full),
            pl.BlockSpec((M, D), full),
            pl.BlockSpec((M, D), full),
            pl.BlockSpec((M, D), full),
        ],
        out_specs=pl.BlockSpec((bB, D), row),
        out_shape=jax.ShapeDtypeStruct((B, D), jnp.float32),
        scratch_shapes=[
            pltpu.VMEM((D, D), jnp.bfloat16),
            pltpu.VMEM((D, H), jnp.bfloat16),
            pltpu.VMEM((M, D), jnp.bfloat16),
            pltpu.VMEM((M, D), jnp.bfloat16),
            pltpu.VMEM((M, D), jnp.bfloat16),
            pltpu.VMEM((M, D), jnp.bfloat16),
            pltpu.VMEM((1, M), jnp.float32),
            pltpu.VMEM((1, M), jnp.float32),
        ],
        compiler_params=pltpu.CompilerParams(
            dimension_semantics=("arbitrary",),
        ),
    )(x, query_w, qb2, gw2, gbm, conf_w1, cb1_2, cw2_2, cb2_2,
      fast_keys, fast_vals, deep_keys, deep_vals)
    return out


# speedup vs baseline: 1.9344x; 1.1929x over previous
import jax
import jax.numpy as jnp
from jax.experimental import pallas as pl
from jax.experimental.pallas import tpu as pltpu

_BLOCK_B = 512


def _inv_norm_rows(k):
    kf = k.astype(jnp.float32)
    n = jnp.sqrt(jnp.sum(kf * kf, axis=1, keepdims=True))
    return 1.0 / jnp.maximum(n, 1e-12)


def _fused_body(x_ref, qw_ref, qb_ref, gw_ref, gbm_ref, cw1_ref, cb1_ref,
                cw2_ref, cb2_ref, fk_ref, fv_ref, dk_ref, dv_ref, out_ref,
                w16, fk16, fv16, dk16, dv16, fkinv, dkinv):
    D = qw_ref.shape[0]
    H = cw1_ref.shape[1]

    @pl.when(pl.program_id(0) == 0)
    def _init():
        w16[:, :D] = qw_ref[:].astype(jnp.bfloat16)
        w16[:, D:D + H] = cw1_ref[:].astype(jnp.bfloat16)
        w16[:, D + H:] = jnp.broadcast_to(
            gw_ref[:].astype(jnp.bfloat16).T, (D, 128))
        fk16[:] = fk_ref[:].astype(jnp.bfloat16)
        fv16[:] = fv_ref[:].astype(jnp.bfloat16)
        dk16[:] = dk_ref[:].astype(jnp.bfloat16)
        dv16[:] = dv_ref[:].astype(jnp.bfloat16)
        fkinv[:] = _inv_norm_rows(fk_ref[:]).T
        dkinv[:] = _inv_norm_rows(dk_ref[:]).T

    xb = x_ref[:].astype(jnp.bfloat16)

    p = jnp.dot(xb, w16[:], preferred_element_type=jnp.float32)
    q = p[:, :D] + qb_ref[:]
    h = jax.nn.relu(p[:, D:D + H] + cb1_ref[:])
    gate = jax.nn.sigmoid(p[:, D + H:D + H + 1] + gbm_ref[:])

    qnorm = jnp.sqrt(jnp.sum(q * q, axis=1, keepdims=True))
    qn = (q * (1.0 / jnp.maximum(qnorm, 1e-12))).astype(jnp.bfloat16)

    def tier(k16, v16, kinv):
        s = jax.lax.dot_general(qn, k16[:], (((1,), (1,)), ((), ())),
                                preferred_element_type=jnp.float32)
        e = jnp.exp(s * kinv[:])
        w = (e * (1.0 / jnp.sum(e, axis=1, keepdims=True))).astype(jnp.bfloat16)
        return jnp.dot(w, v16[:], preferred_element_type=jnp.float32)

    fast_out = tier(fk16, fv16, fkinv)
    deep_out = tier(dk16, dv16, dkinv)

    c = jnp.sum(h * cw2_ref[:], axis=1, keepdims=True) + cb2_ref[:]
    conf = jax.nn.sigmoid(c)

    out_ref[:] = (deep_out + gate * (fast_out - deep_out)) * conf


def kernel(x, query_w, query_b, gate_w, gate_b, mix_logit, conf_w1, conf_b1,
           conf_w2, conf_b2, fast_keys, fast_vals, deep_keys, deep_vals):
    B, D = x.shape
    M = fast_keys.shape[0]
    H = conf_w1.shape[1]
    bB = _BLOCK_B

    qb2 = query_b.reshape(1, D)
    gw2 = gate_w.reshape(1, D)
    gbm = (gate_b + mix_logit).reshape(1, 1)
    cb1_2 = conf_b1.reshape(1, H)
    cw2_2 = conf_w2.reshape(1, H)
    cb2_2 = conf_b2.reshape(1, 1)

    full = lambda i: (0, 0)
    row = lambda i: (i, 0)

    out = pl.pallas_call(
        _fused_body,
        grid=(B // bB,),
        in_specs=[
            pl.BlockSpec((bB, D), row),
            pl.BlockSpec((D, D), full),
            pl.BlockSpec((1, D), full),
            pl.BlockSpec((1, D), full),
            pl.BlockSpec((1, 1), full),
            pl.BlockSpec((D, H), full),
            pl.BlockSpec((1, H), full),
            pl.BlockSpec((1, H), full),
            pl.BlockSpec((1, 1), full),
            pl.BlockSpec((M, D), full),
            pl.BlockSpec((M, D), full),
            pl.BlockSpec((M, D), full),
            pl.BlockSpec((M, D), full),
        ],
        out_specs=pl.BlockSpec((bB, D), row),
        out_shape=jax.ShapeDtypeStruct((B, D), jnp.float32),
        scratch_shapes=[
            pltpu.VMEM((D, D + H + 128), jnp.bfloat16),
            pltpu.VMEM((M, D), jnp.bfloat16),
            pltpu.VMEM((M, D), jnp.bfloat16),
            pltpu.VMEM((M, D), jnp.bfloat16),
            pltpu.VMEM((M, D), jnp.bfloat16),
            pltpu.VMEM((1, M), jnp.float32),
            pltpu.VMEM((1, M), jnp.float32),
        ],
        compiler_params=pltpu.CompilerParams(
            dimension_semantics=("arbitrary",),
        ),
    )(x, query_w, qb2, gw2, gbm, conf_w1, cb1_2, cw2_2, cb2_2,
      fast_keys, fast_vals, deep_keys, deep_vals)
    return out


# speedup vs baseline: 2.0103x; 1.0392x over previous
import jax
import jax.numpy as jnp
from jax.experimental import pallas as pl
from jax.experimental.pallas import tpu as pltpu

_BLOCK_B = 512


def _inv_norm_rows(k):
    kf = k.astype(jnp.float32)
    n = jnp.sqrt(jnp.sum(kf * kf, axis=1, keepdims=True))
    return 1.0 / jnp.maximum(n, 1e-12)


def _fused_body(x_ref, qw_ref, qb_ref, gw_ref, gbm_ref, cw1_ref, cb1_ref,
                cw2_ref, cb2_ref, fk_ref, fv_ref, dk_ref, dv_ref, out_ref,
                w16, k2, v2, kinv2):
    D = qw_ref.shape[0]
    H = cw1_ref.shape[1]
    M = fk_ref.shape[0]

    @pl.when(pl.program_id(0) == 0)
    def _init():
        w16[:, :D] = qw_ref[:].astype(jnp.bfloat16)
        w16[:, D:D + H] = cw1_ref[:].astype(jnp.bfloat16)
        w16[:, D + H:] = jnp.broadcast_to(
            gw_ref[:].astype(jnp.bfloat16).T, (D, 128))
        k2[:M] = fk_ref[:].astype(jnp.bfloat16)
        k2[M:] = dk_ref[:].astype(jnp.bfloat16)
        v2[:M] = fv_ref[:].astype(jnp.bfloat16)
        v2[M:] = dv_ref[:].astype(jnp.bfloat16)
        kinv2[:, :M] = _inv_norm_rows(fk_ref[:]).T
        kinv2[:, M:] = _inv_norm_rows(dk_ref[:]).T

    xb = x_ref[:].astype(jnp.bfloat16)

    p = jnp.dot(xb, w16[:], preferred_element_type=jnp.float32)
    q = p[:, :D] + qb_ref[:]
    h = jax.nn.relu(p[:, D:D + H] + cb1_ref[:])
    gate = jax.nn.sigmoid(p[:, D + H:D + H + 1] + gbm_ref[:])

    qnorm = jnp.sqrt(jnp.sum(q * q, axis=1, keepdims=True))
    qn = (q * (1.0 / jnp.maximum(qnorm, 1e-12))).astype(jnp.bfloat16)

    s = jax.lax.dot_general(qn, k2[:], (((1,), (1,)), ((), ())),
                            preferred_element_type=jnp.float32)
    e = jnp.exp(s * kinv2[:])
    sf = jnp.sum(e[:, :M], axis=1, keepdims=True)
    sd = jnp.sum(e[:, M:], axis=1, keepdims=True)

    c = jnp.sum(h * cw2_ref[:], axis=1, keepdims=True) + cb2_ref[:]
    conf = jax.nn.sigmoid(c)

    af = gate * conf * (1.0 / sf)
    ad = (1.0 - gate) * conf * (1.0 / sd)
    w2 = jnp.concatenate([e[:, :M] * af, e[:, M:] * ad],
                         axis=1).astype(jnp.bfloat16)
    out_ref[:] = jnp.dot(w2, v2[:], preferred_element_type=jnp.float32)


def kernel(x, query_w, query_b, gate_w, gate_b, mix_logit, conf_w1, conf_b1,
           conf_w2, conf_b2, fast_keys, fast_vals, deep_keys, deep_vals):
    B, D = x.shape
    M = fast_keys.shape[0]
    H = conf_w1.shape[1]
    bB = _BLOCK_B

    qb2 = query_b.reshape(1, D)
    gw2 = gate_w.reshape(1, D)
    gbm = (gate_b + mix_logit).reshape(1, 1)
    cb1_2 = conf_b1.reshape(1, H)
    cw2_2 = conf_w2.reshape(1, H)
    cb2_2 = conf_b2.reshape(1, 1)

    full = lambda i: (0, 0)
    row = lambda i: (i, 0)

    out = pl.pallas_call(
        _fused_body,
        grid=(B // bB,),
        in_specs=[
            pl.BlockSpec((bB, D), row),
            pl.BlockSpec((D, D), full),
            pl.BlockSpec((1, D), full),
            pl.BlockSpec((1, D), full),
            pl.BlockSpec((1, 1), full),
            pl.BlockSpec((D, H), full),
            pl.BlockSpec((1, H), full),
            pl.BlockSpec((1, H), full),
            pl.BlockSpec((1, 1), full),
            pl.BlockSpec((M, D), full),
            pl.BlockSpec((M, D), full),
            pl.BlockSpec((M, D), full),
            pl.BlockSpec((M, D), full),
        ],
        out_specs=pl.BlockSpec((bB, D), row),
        out_shape=jax.ShapeDtypeStruct((B, D), jnp.float32),
        scratch_shapes=[
            pltpu.VMEM((D, D + H + 128), jnp.bfloat16),
            pltpu.VMEM((2 * M, D), jnp.bfloat16),
            pltpu.VMEM((2 * M, D), jnp.bfloat16),
            pltpu.VMEM((1, 2 * M), jnp.float32),
        ],
        compiler_params=pltpu.CompilerParams(
            dimension_semantics=("arbitrary",),
        ),
    )(x, query_w, qb2, gw2, gbm, conf_w1, cb1_2, cw2_2, cb2_2,
      fast_keys, fast_vals, deep_keys, deep_vals)
    return out
